# baseline (speedup 1.0000x reference)
import functools

import jax
import jax.numpy as jnp
from jax import lax
from jax.experimental import pallas as pl
from jax.experimental.pallas import tpu as pltpu
from jax.experimental.pallas import tpu_sc as plsc

NC = 2
NS = 16
NW = NC * NS
L = 16

EPS = 1e-12


def _rsqrt(x):
    xi = lax.bitcast_convert_type(x, jnp.int32)
    yi = jnp.int32(0x5F3759DF) - (xi >> 1)
    y = lax.bitcast_convert_type(yi, jnp.float32)
    for _ in range(3):
        y = y * (1.5 - 0.5 * x * y * y)
    return y


def _make_sc_kernel(n_tok, hidden, seq):
    p_per_w = seq // NW
    n_batch = n_tok // seq
    n_chunk = hidden // L
    inv_h = 1.0 / hidden

    mesh = plsc.VectorSubcoreMesh(core_axis_name="c", subcore_axis_name="s")

    @functools.partial(
        pl.kernel,
        out_type=jax.ShapeDtypeStruct((n_tok, hidden), jnp.float32),
        mesh=mesh,
        compiler_params=pltpu.CompilerParams(needs_layout_passes=False),
        scratch_types=[
            pltpu.VMEM((p_per_w,), jnp.int32),
            pltpu.VMEM((p_per_w, hidden), jnp.float32),
            pltpu.VMEM((p_per_w, hidden), jnp.float32),
            pltpu.VMEM((hidden,), jnp.float32),
            pltpu.VMEM((hidden,), jnp.float32),
            pltpu.SemaphoreType.DMA,
        ],
    )
    def sc_kernel(ids_hbm, word_hbm, pos_hbm, gam_hbm, bet_hbm, out_hbm,
                  idx_v, rows_v, pos_v, gam_v, bet_v, sem):
        wid = lax.axis_index("s") * NC + lax.axis_index("c")
        pcol = wid * p_per_w

        pltpu.sync_copy(pos_hbm.at[pl.ds(pcol, p_per_w)], pos_v)
        pltpu.sync_copy(gam_hbm, gam_v)
        pltpu.sync_copy(bet_hbm, bet_v)

        def batch_body(b, carry):
            base = b * seq + pcol
            pltpu.sync_copy(ids_hbm.at[pl.ds(base, p_per_w)], idx_v)
            pltpu.async_copy(word_hbm.at[idx_v], rows_v, sem).wait()

            def row_body(r, carry2):
                acc_s = jnp.zeros((L,), jnp.float32)
                acc_q = jnp.zeros((L,), jnp.float32)
                for j in range(n_chunk):
                    sl = pl.ds(j * L, L)
                    v = rows_v[r, sl] + pos_v[r, sl]
                    rows_v[r, sl] = v
                    acc_s = acc_s + v
                    acc_q = acc_q + v * v
                s = jnp.sum(acc_s)
                q = jnp.sum(acc_q)
                mean = s * inv_h
                var = q * inv_h - mean * mean
                rstd = _rsqrt(var + EPS)
                mean_bc = jnp.full((L,), mean, jnp.float32)
                rstd_bc = jnp.full((L,), rstd, jnp.float32)
                for j in range(n_chunk):
                    sl = pl.ds(j * L, L)
                    t = gam_v[sl] * rstd_bc
                    u = bet_v[sl] - mean_bc * t
                    rows_v[r, sl] = rows_v[r, sl] * t + u
                return carry2

            lax.fori_loop(0, p_per_w, row_body, 0)
            pltpu.sync_copy(rows_v, out_hbm.at[pl.ds(base, p_per_w)])
            return carry

        lax.fori_loop(0, n_batch, batch_body, 0)

    return sc_kernel


def kernel(input_ids, word_emb, pos_emb, gamma, beta):
    batch, seq = input_ids.shape
    hidden = word_emb.shape[1]
    n_tok = batch * seq
    ids = input_ids.reshape(n_tok).astype(jnp.int32)
    sc = _make_sc_kernel(n_tok, hidden, seq)
    out = sc(ids, word_emb, pos_emb, gamma, beta)
    return out.reshape(batch, seq, hidden)


# speedup vs baseline: 2.9211x; 2.9211x over previous
import functools

import jax
import jax.numpy as jnp
from jax import lax
from jax.experimental import pallas as pl
from jax.experimental.pallas import tpu as pltpu
from jax.experimental.pallas import tpu_sc as plsc

NC = 2
NS = 16
NW = NC * NS
L = 16

EPS = 1e-12


def _rsqrt_vec(x):
    xi = plsc.bitcast(x, jnp.int32)
    yi = jnp.int32(0x5F3759DF) - (xi >> 1)
    y = plsc.bitcast(yi, jnp.float32)
    for _ in range(3):
        y = y * (1.5 - 0.5 * x * y * y)
    return y


def _make_sc_kernel(n_batch, seq, hidden):
    p_per_w = seq // NW
    n_chunk = hidden // L
    inv_h = 1.0 / hidden
    n_tok = n_batch * seq

    mesh = plsc.VectorSubcoreMesh(core_axis_name="c", subcore_axis_name="s")

    @functools.partial(
        pl.kernel,
        out_type=jax.ShapeDtypeStruct((n_tok, hidden), jnp.float32),
        mesh=mesh,
        compiler_params=pltpu.CompilerParams(needs_layout_passes=False),
        scratch_types=[
            pltpu.VMEM((n_batch * p_per_w,), jnp.int32),
            pltpu.VMEM((p_per_w, hidden), jnp.float32),
            pltpu.VMEM((p_per_w, hidden), jnp.float32),
            pltpu.VMEM((p_per_w, hidden), jnp.float32),
            pltpu.VMEM((p_per_w, hidden), jnp.float32),
            pltpu.VMEM((p_per_w, hidden), jnp.float32),
            pltpu.VMEM((p_per_w, L), jnp.float32),
            pltpu.VMEM((p_per_w, L), jnp.float32),
            pltpu.VMEM((hidden,), jnp.float32),
            pltpu.VMEM((hidden,), jnp.float32),
            pltpu.SemaphoreType.DMA,
            pltpu.SemaphoreType.DMA,
            pltpu.SemaphoreType.DMA,
            pltpu.SemaphoreType.DMA,
        ],
    )
    def sc_kernel(ids_hbm, word_hbm, pos_hbm, gam_hbm, bet_hbm, out_hbm,
                  idx_all, rows0, rows1, obuf0, obuf1, pos_v,
                  stat_m, stat_r, gam_v, bet_v,
                  gsem0, gsem1, ssem0, ssem1):
        wid = lax.axis_index("s") * NC + lax.axis_index("c")
        pcol = wid * p_per_w

        pltpu.sync_copy(ids_hbm.at[pl.ds(wid * (n_batch * p_per_w),
                                         n_batch * p_per_w)], idx_all)
        pltpu.sync_copy(pos_hbm.at[pl.ds(pcol, p_per_w)], pos_v)
        pltpu.sync_copy(gam_hbm, gam_v)
        pltpu.sync_copy(bet_hbm, bet_v)

        rows = (rows0, rows1)
        obuf = (obuf0, obuf1)
        gsem = (gsem0, gsem1)
        ssem = (ssem0, ssem1)

        def gather_start(b, slot):
            idx = idx_all.at[pl.ds(b * p_per_w, p_per_w)]
            pltpu.async_copy(word_hbm.at[idx], rows[slot], gsem[slot])

        def gather_wait(b, slot):
            idx = idx_all.at[pl.ds(b * p_per_w, p_per_w)]
            pltpu.make_async_copy(word_hbm.at[idx], rows[slot],
                                  gsem[slot]).wait()

        def store_start(b, slot):
            base = b * seq + pcol
            pltpu.async_copy(obuf[slot],
                             out_hbm.at[pl.ds(base, p_per_w)], ssem[slot])

        def store_wait(b, slot):
            base = b * seq + pcol
            pltpu.make_async_copy(obuf[slot],
                                  out_hbm.at[pl.ds(base, p_per_w)],
                                  ssem[slot]).wait()

        def compute(slot):
            rows_ref = rows[slot]
            obuf_ref = obuf[slot]

            def row_stats(r, carry):
                acc_s = jnp.zeros((L,), jnp.float32)
                acc_q = jnp.zeros((L,), jnp.float32)
                for j in range(n_chunk):
                    sl = pl.ds(j * L, L)
                    v = rows_ref[r, sl] + pos_v[r, sl]
                    rows_ref[r, sl] = v
                    acc_s = acc_s + v
                    acc_q = acc_q + v * v
                mean = jnp.sum(acc_s) * inv_h
                var = jnp.sum(acc_q) * inv_h - mean * mean
                stat_m[r] = jnp.full((L,), mean, jnp.float32)
                stat_r[r] = _rsqrt_vec(jnp.full((L,), var + EPS, jnp.float32))
                return carry

            lax.fori_loop(0, p_per_w, row_stats, 0)

            ms = [stat_m[r] for r in range(p_per_w)]
            rs = [stat_r[r] for r in range(p_per_w)]

            def norm_chunk(j, carry):
                sl = pl.ds(j * L, L)
                g = gam_v[sl]
                bb = bet_v[sl]
                for r in range(p_per_w):
                    v = rows_ref[r, sl]
                    obuf_ref[r, sl] = (v - ms[r]) * rs[r] * g + bb
                return carry

            lax.fori_loop(0, n_chunk, norm_chunk, 0)

        gather_start(0, 0)
        gather_start(1, 1)

        @pl.loop(0, n_batch, step=2)
        def batch_loop(i):
            for k in range(2):
                b = i + k
                slot = k

                @pl.when(b >= 2)
                def _():
                    store_wait(b - 2, slot)

                gather_wait(b, slot)
                compute(slot)

                @pl.when(b + 2 < n_batch)
                def _():
                    gather_start(b + 2, slot)

                store_start(b, slot)

        store_wait(n_batch - 2, 0)
        store_wait(n_batch - 1, 1)

    return sc_kernel


def kernel(input_ids, word_emb, pos_emb, gamma, beta):
    batch, seq = input_ids.shape
    hidden = word_emb.shape[1]
    p_per_w = seq // NW
    ids = (input_ids.astype(jnp.int32)
           .reshape(batch, NW, p_per_w)
           .swapaxes(0, 1)
           .reshape(batch * seq))
    sc = _make_sc_kernel(batch, seq, hidden)
    out = sc(ids, word_emb, pos_emb, gamma, beta)
    return out.reshape(batch, seq, hidden)


# speedup vs baseline: 3.8995x; 1.3349x over previous
import functools

import jax
import jax.numpy as jnp
from jax import lax
from jax.experimental import pallas as pl
from jax.experimental.pallas import tpu as pltpu
from jax.experimental.pallas import tpu_sc as plsc

NC = 2
NS = 16
NW = NC * NS
L = 16

EPS = 1e-12


def _rsqrt_vec(x):
    xi = plsc.bitcast(x, jnp.int32)
    yi = jnp.int32(0x5F3759DF) - (xi >> 1)
    y = plsc.bitcast(yi, jnp.float32)
    for _ in range(3):
        y = y * (1.5 - 0.5 * x * y * y)
    return y


def _make_sc_kernel(n_batch, seq, hidden):
    p_per_w = seq // NW
    n_chunk = hidden // L
    inv_h = 1.0 / hidden
    n_tok = n_batch * seq

    mesh = plsc.VectorSubcoreMesh(core_axis_name="c", subcore_axis_name="s")

    @functools.partial(
        pl.kernel,
        out_type=jax.ShapeDtypeStruct((n_tok, hidden), jnp.float32),
        mesh=mesh,
        compiler_params=pltpu.CompilerParams(needs_layout_passes=False),
        scratch_types=[
            pltpu.VMEM((n_batch * p_per_w,), jnp.int32),
            pltpu.VMEM((p_per_w, hidden), jnp.float32),
            pltpu.VMEM((p_per_w, hidden), jnp.float32),
            pltpu.VMEM((p_per_w, hidden), jnp.float32),
            pltpu.VMEM((p_per_w, hidden), jnp.float32),
            pltpu.VMEM((p_per_w, hidden), jnp.float32),
            pltpu.VMEM((p_per_w, L), jnp.float32),
            pltpu.VMEM((p_per_w, L), jnp.float32),
            pltpu.VMEM((hidden,), jnp.float32),
            pltpu.VMEM((hidden,), jnp.float32),
            pltpu.SemaphoreType.DMA,
            pltpu.SemaphoreType.DMA,
            pltpu.SemaphoreType.DMA,
            pltpu.SemaphoreType.DMA,
        ],
    )
    def sc_kernel(ids_hbm, word_hbm, pos_hbm, gam_hbm, bet_hbm, out_hbm,
                  idx_all, rows0, rows1, obuf0, obuf1, pos_v,
                  stat_m, stat_r, gam_v, bet_v,
                  gsem0, gsem1, ssem0, ssem1):
        wid = lax.axis_index("s") * NC + lax.axis_index("c")
        pcol = wid * p_per_w

        pltpu.sync_copy(ids_hbm.at[pl.ds(wid * (n_batch * p_per_w),
                                         n_batch * p_per_w)], idx_all)
        pltpu.sync_copy(pos_hbm.at[pl.ds(pcol, p_per_w)], pos_v)
        pltpu.sync_copy(gam_hbm, gam_v)
        pltpu.sync_copy(bet_hbm, bet_v)

        rows = (rows0, rows1)
        obuf = (obuf0, obuf1)
        gsem = (gsem0, gsem1)
        ssem = (ssem0, ssem1)

        def gather_start(b, slot):
            idx = idx_all.at[pl.ds(b * p_per_w, p_per_w)]
            pltpu.async_copy(word_hbm.at[idx], rows[slot], gsem[slot])

        def gather_wait(b, slot):
            idx = idx_all.at[pl.ds(b * p_per_w, p_per_w)]
            pltpu.make_async_copy(word_hbm.at[idx], rows[slot],
                                  gsem[slot]).wait()

        def store_start(b, slot):
            base = b * seq + pcol
            pltpu.async_copy(obuf[slot],
                             out_hbm.at[pl.ds(base, p_per_w)], ssem[slot])

        def store_wait(b, slot):
            base = b * seq + pcol
            pltpu.make_async_copy(obuf[slot],
                                  out_hbm.at[pl.ds(base, p_per_w)],
                                  ssem[slot]).wait()

        def compute(slot):
            rows_ref = rows[slot]
            obuf_ref = obuf[slot]

            @plsc.parallel_loop(0, p_per_w, unroll=2)
            def row_stats(r):
                acc_s = jnp.zeros((L,), jnp.float32)
                acc_q = jnp.zeros((L,), jnp.float32)
                for j in range(n_chunk):
                    sl = pl.ds(j * L, L)
                    v = rows_ref[r, sl] + pos_v[r, sl]
                    rows_ref[r, sl] = v
                    acc_s = acc_s + v
                    acc_q = acc_q + v * v
                mean = jnp.sum(acc_s) * inv_h
                var = jnp.sum(acc_q) * inv_h - mean * mean
                rstd = _rsqrt_vec(jnp.full((L,), var + EPS, jnp.float32))
                stat_r[r] = rstd
                stat_m[r] = mean * rstd

            cs = [stat_m[r] for r in range(p_per_w)]
            rs = [stat_r[r] for r in range(p_per_w)]

            @plsc.parallel_loop(0, n_chunk)
            def norm_chunk(j):
                sl = pl.ds(j * L, L)
                g = gam_v[sl]
                bb = bet_v[sl]
                for r in range(p_per_w):
                    v = rows_ref[r, sl]
                    obuf_ref[r, sl] = (v * rs[r] - cs[r]) * g + bb

        gather_start(0, 0)
        gather_start(1, 1)

        @pl.loop(0, n_batch, step=2)
        def batch_loop(i):
            for k in range(2):
                b = i + k
                slot = k

                @pl.when(b >= 2)
                def _():
                    store_wait(b - 2, slot)

                gather_wait(b, slot)
                compute(slot)

                @pl.when(b + 2 < n_batch)
                def _():
                    gather_start(b + 2, slot)

                store_start(b, slot)

        store_wait(n_batch - 2, 0)
        store_wait(n_batch - 1, 1)

    return sc_kernel


def kernel(input_ids, word_emb, pos_emb, gamma, beta):
    batch, seq = input_ids.shape
    hidden = word_emb.shape[1]
    p_per_w = seq // NW
    ids = (input_ids.astype(jnp.int32)
           .reshape(batch, NW, p_per_w)
           .swapaxes(0, 1)
           .reshape(batch * seq))
    sc = _make_sc_kernel(batch, seq, hidden)
    out = sc(ids, word_emb, pos_emb, gamma, beta)
    return out.reshape(batch, seq, hidden)
